# baseline (speedup 1.0000x reference)
import functools
import math

import jax
import jax.numpy as jnp
from jax import lax
from jax.experimental import pallas as pl
from jax.experimental.pallas import tpu as pltpu
from jax.experimental.pallas import tpu_sc as plsc

N = 4096
G = 256
H = 128
F = 128
NG = 50
T = 6
CUTOFF = 5.0
E_MAX = 131072

LOG2 = math.log(2.0)
_DELTA = CUTOFF / (NG - 1)
_COEFF = -0.5 / _DELTA**2

ROWS = 512
ET = 512
NROW_BLK = N // ROWS
NE_BLK = E_MAX // ET

SC_TILES = 32
EPW = E_MAX // SC_TILES
CK = 128
NCH = EPW // CK


def _ssp(x):
    return jnp.maximum(x, 0.0) + jnp.log(1.0 + jnp.exp(-jnp.abs(x))) - LOG2


def _sigmoid(x):
    return 1.0 / (1.0 + jnp.exp(-x))


def _embed_body(z_ref, emb_ref, cf1_ref, h_ref, xp_ref):
    z = z_ref[0, 0, :].reshape(ROWS, 1)
    cols = lax.broadcasted_iota(jnp.int32, (ROWS, 128), 1)
    onehot = (z == cols).astype(jnp.float32)
    h = jnp.dot(onehot, emb_ref[...], preferred_element_type=jnp.float32)
    h_ref[...] = h
    xp_ref[...] = jnp.dot(h, cf1_ref[...], preferred_element_type=jnp.float32)


def _embed(z3, emb_p, cf1_0):
    return pl.pallas_call(
        _embed_body,
        grid=(NROW_BLK,),
        in_specs=[
            pl.BlockSpec((1, 1, ROWS), lambda i: (i, 0, 0)),
            pl.BlockSpec((128, H), lambda i: (0, 0)),
            pl.BlockSpec((H, H), lambda i: (0, 0)),
        ],
        out_specs=[
            pl.BlockSpec((ROWS, H), lambda i: (i, 0)),
            pl.BlockSpec((ROWS, H), lambda i: (i, 0)),
        ],
        out_shape=[
            jax.ShapeDtypeStruct((N, H), jnp.float32),
            jax.ShapeDtypeStruct((N, H), jnp.float32),
        ],
    )(z3, emb_p, cf1_0)


def _pair_body(offs_ref, tbl_ref, srcf_ref, dstf_ref, vm_ref):
    i = pl.program_id(0)
    k = i * ET + lax.broadcasted_iota(jnp.int32, (1, ET), 1)
    kf = k.astype(jnp.float32)
    cmp = (offs_ref[...] <= kf).astype(jnp.float32)
    ge = jnp.sum(cmp, axis=0, keepdims=True).astype(jnp.int32) - 1
    vm = ge < G
    gec = jnp.clip(ge, 0, G - 1)
    onehot = (gec == lax.broadcasted_iota(jnp.int32, (G, ET), 0))
    look = jnp.dot(tbl_ref[...], onehot.astype(jnp.float32),
                   preferred_element_type=jnp.float32,
                   precision=lax.Precision.HIGHEST)
    offs_e = look[0:1, :].astype(jnp.int32)
    sego_e = look[1:2, :].astype(jnp.int32)
    sm1f = look[2:3, :]
    sm1 = sm1f.astype(jnp.int32)
    local = k - offs_e
    a = jnp.floor(local.astype(jnp.float32) / sm1f).astype(jnp.int32)
    for _ in range(2):
        r = local - a * sm1
        a = a - (r < 0).astype(jnp.int32) + (r >= sm1).astype(jnp.int32)
    r = local - a * sm1
    b = r + (r >= a).astype(jnp.int32)
    srcf_ref[...] = jnp.where(vm, sego_e + a, N).reshape(1, 1, ET)
    dstf_ref[...] = jnp.where(vm, sego_e + b, 0).reshape(1, 1, ET)
    vm_ref[...] = vm.astype(jnp.float32).reshape(1, 1, ET)


def _pair_enum(offs_col, tbl_t):
    return pl.pallas_call(
        _pair_body,
        grid=(NE_BLK,),
        in_specs=[
            pl.BlockSpec((384, 1), lambda i: (0, 0)),
            pl.BlockSpec((8, G), lambda i: (0, 0)),
        ],
        out_specs=[
            pl.BlockSpec((1, 1, ET), lambda i: (i, 0, 0)),
            pl.BlockSpec((1, 1, ET), lambda i: (i, 0, 0)),
            pl.BlockSpec((1, 1, ET), lambda i: (i, 0, 0)),
        ],
        out_shape=[
            jax.ShapeDtypeStruct((NE_BLK, 1, ET), jnp.int32),
            jax.ShapeDtypeStruct((NE_BLK, 1, ET), jnp.int32),
            jax.ShapeDtypeStruct((NE_BLK, 1, ET), jnp.float32),
        ],
    )(offs_col, tbl_t)


def _scg_body(px_hbm, py_hbm, pz_hbm, srcf_hbm, dstf_hbm, d2_hbm,
              sv, dv, siv, ax, ay, az, bx, by, bz, d2v, sem):
    c = lax.axis_index("c")
    s = lax.axis_index("s")
    tile = c * 16 + s
    base0 = tile * EPW
    nmax = jnp.full((16,), N - 1, jnp.int32)

    def chunk(j, carry):
        base = base0 + j * CK
        pltpu.sync_copy(srcf_hbm.at[pl.ds(base, CK)], sv)
        pltpu.sync_copy(dstf_hbm.at[pl.ds(base, CK)], dv)

        def clip_row(q, cc):
            sl = pl.ds(q * 16, 16)
            siv[sl] = jnp.minimum(sv[sl], nmax)
            return cc

        lax.fori_loop(0, CK // 16, clip_row, 0)
        cps = [pltpu.async_copy(px_hbm.at[siv], ax, sem),
               pltpu.async_copy(py_hbm.at[siv], ay, sem),
               pltpu.async_copy(pz_hbm.at[siv], az, sem),
               pltpu.async_copy(px_hbm.at[dv], bx, sem),
               pltpu.async_copy(py_hbm.at[dv], by, sem),
               pltpu.async_copy(pz_hbm.at[dv], bz, sem)]
        for cp in cps:
            cp.wait()

        def d2_row(q, cc):
            sl = pl.ds(q * 16, 16)
            dx = ax[sl] - bx[sl]
            dy = ay[sl] - by[sl]
            dz = az[sl] - bz[sl]
            d2v[sl] = dx * dx + dy * dy + dz * dz
            return cc

        lax.fori_loop(0, CK // 16, d2_row, 0)
        pltpu.sync_copy(d2v, d2_hbm.at[pl.ds(base, CK)])
        return carry

    lax.fori_loop(0, NCH, chunk, 0)


def _sc_edge_d2(px, py, pz, srcf, dstf):
    mesh = plsc.VectorSubcoreMesh(core_axis_name="c", subcore_axis_name="s")
    fn = pl.kernel(
        _scg_body,
        out_type=jax.ShapeDtypeStruct((E_MAX,), jnp.float32),
        mesh=mesh,
        scratch_types=[
            pltpu.VMEM((CK,), jnp.int32),
            pltpu.VMEM((CK,), jnp.int32),
            pltpu.VMEM((CK,), jnp.int32),
            pltpu.VMEM((CK,), jnp.float32),
            pltpu.VMEM((CK,), jnp.float32),
            pltpu.VMEM((CK,), jnp.float32),
            pltpu.VMEM((CK,), jnp.float32),
            pltpu.VMEM((CK,), jnp.float32),
            pltpu.VMEM((CK,), jnp.float32),
            pltpu.VMEM((CK,), jnp.float32),
            pltpu.SemaphoreType.DMA,
        ],
    )
    return fn(px, py, pz, srcf, dstf)


def _filter_body(d2_ref, vm_ref, w1_ref, b1_ref, w2_ref, b2_ref, w_ref):
    d2 = d2_ref[0, 0, :].reshape(ET, 1)
    d = jnp.sqrt(d2 + 1e-12)
    valid = vm_ref[0, 0, :].reshape(ET, 1) * \
        (d2 < CUTOFF**2).astype(jnp.float32)
    c = 0.5 * (jnp.cos(d * (math.pi / CUTOFF)) + 1.0) * valid
    k = lax.broadcasted_iota(jnp.int32, (ET, 64), 1)
    off = k.astype(jnp.float32) * _DELTA
    rbf = jnp.exp(_COEFF * (d - off) ** 2)
    rbf = jnp.where(k < NG, rbf, 0.0)
    a = _ssp(jnp.dot(rbf, w1_ref[...], preferred_element_type=jnp.float32)
             + b1_ref[...])
    w = jnp.dot(a, w2_ref[...], preferred_element_type=jnp.float32) + b2_ref[...]
    w_ref[...] = w * c


def _edge_filter(d3, c3, w1_p, b1, w2, b2):
    return pl.pallas_call(
        _filter_body,
        grid=(NE_BLK,),
        in_specs=[
            pl.BlockSpec((1, 1, ET), lambda i: (i, 0, 0)),
            pl.BlockSpec((1, 1, ET), lambda i: (i, 0, 0)),
            pl.BlockSpec((64, F), lambda i: (0, 0)),
            pl.BlockSpec((1, F), lambda i: (0, 0)),
            pl.BlockSpec((F, H), lambda i: (0, 0)),
            pl.BlockSpec((1, H), lambda i: (0, 0)),
        ],
        out_specs=pl.BlockSpec((ET, H), lambda i: (i, 0)),
        out_shape=jax.ShapeDtypeStruct((E_MAX, H), jnp.float32),
    )(d3, c3, w1_p, b1, w2, b2)


SRPT = N // SC_TILES


def _sc_body(xp_hbm, w_hbm, srcf_hbm, dstf_hbm, cb_hbm, out_hbm,
             cbv, sv, dv, wv, xv, agg, sem, semi):
    c = lax.axis_index("c")
    s = lax.axis_index("s")
    tile = c * 16 + s
    lo = tile * SRPT
    zero = jnp.zeros((16,), jnp.float32)
    iotas = [lax.iota(jnp.int32, 16) + g * 16 for g in range(H // 16)]

    def zrow(r, carry):
        for g in range(H // 16):
            agg[pl.ds(r * H + g * 16, 16)] = zero
        return carry

    lax.fori_loop(0, SRPT, zrow, 0)

    pltpu.sync_copy(cb_hbm.at[tile], cbv)
    cbl = cbv[...]
    c0 = cbl[0]
    c1 = cbl[1]

    def chunk(j, carry):
        base = j * CK
        pltpu.sync_copy(dstf_hbm.at[pl.ds(base, CK)], dv)
        pltpu.sync_copy(srcf_hbm.at[pl.ds(base, CK)], sv.at[pl.ds(0, CK)])
        sv[pl.ds(CK, 16)] = jnp.full((16,), N, jnp.int32)
        cp_w = pltpu.async_copy(w_hbm.at[pl.ds(base, CK)], wv, sem)
        cp_x = pltpu.async_copy(xp_hbm.at[dv], xv, semi)
        cp_w.wait()
        cp_x.wait()

        def edge(e, cc):
            row = sv[pl.ds(e, 16)][0] - lo

            @pl.when((row >= 0) & (row < SRPT))
            def _():
                base = row * H
                for g in range(H // 16):
                    sl = pl.ds(g * 16, 16)
                    sla = pl.ds(base + g * 16, 16)
                    v = xv[e, sl] * wv[e, sl]
                    agg[sla] = agg[sla] + v

            return cc

        lax.fori_loop(0, CK, edge, 0)
        return carry

    lax.fori_loop(c0, c1, chunk, 0)
    pltpu.sync_copy(agg, out_hbm.at[pl.ds(lo * H, SRPT * H)])


def _sc_edge_messages(xp, w_e, srcf, dstf, cb):
    mesh = plsc.VectorSubcoreMesh(core_axis_name="c", subcore_axis_name="s")
    fn = pl.kernel(
        _sc_body,
        out_type=jax.ShapeDtypeStruct((N * H,), jnp.float32),
        mesh=mesh,
        scratch_types=[
            pltpu.VMEM((16,), jnp.int32),
            pltpu.VMEM((CK + 16,), jnp.int32),
            pltpu.VMEM((CK,), jnp.int32),
            pltpu.VMEM((CK, H), jnp.float32),
            pltpu.VMEM((CK, H), jnp.float32),
            pltpu.VMEM((SRPT * H,), jnp.float32),
            pltpu.SemaphoreType.DMA,
            pltpu.SemaphoreType.DMA,
        ],
    )
    return fn(xp, w_e, srcf, dstf, cb).reshape(N, H)


def _update_body(agg_ref, h_ref, cf2w_ref, cf2b_ref, linw_ref, linb_ref,
                 cf1n_ref, hn_ref, xpn_ref):
    agg = agg_ref[...]
    xc = _ssp(jnp.dot(agg, cf2w_ref[...], preferred_element_type=jnp.float32)
              + cf2b_ref[...])
    xc = jnp.dot(xc, linw_ref[...], preferred_element_type=jnp.float32) \
        + linb_ref[...]
    hn = h_ref[...] + xc
    hn_ref[...] = hn
    xpn_ref[...] = jnp.dot(hn, cf1n_ref[...], preferred_element_type=jnp.float32)


def _update(aggp, h, cf2w, cf2b, linw, linb, cf1n):
    return pl.pallas_call(
        _update_body,
        grid=(NROW_BLK,),
        in_specs=[
            pl.BlockSpec((ROWS, H), lambda i: (i, 0)),
            pl.BlockSpec((ROWS, H), lambda i: (i, 0)),
            pl.BlockSpec((H, F), lambda i: (0, 0)),
            pl.BlockSpec((1, F), lambda i: (0, 0)),
            pl.BlockSpec((H, H), lambda i: (0, 0)),
            pl.BlockSpec((1, H), lambda i: (0, 0)),
            pl.BlockSpec((H, H), lambda i: (0, 0)),
        ],
        out_specs=[
            pl.BlockSpec((ROWS, H), lambda i: (i, 0)),
            pl.BlockSpec((ROWS, H), lambda i: (i, 0)),
        ],
        out_shape=[
            jax.ShapeDtypeStruct((N, H), jnp.float32),
            jax.ShapeDtypeStruct((N, H), jnp.float32),
        ],
    )(aggp, h, cf2w, cf2b, linw, linb, cf1n)


def _out_body(h_ref, b_ref, w1_ref, b1_ref, w2_ref, b2_ref, out_ref):
    i = pl.program_id(0)
    a = _ssp(jnp.dot(h_ref[...], w1_ref[...], preferred_element_type=jnp.float32)
             + b1_ref[...])
    o = jnp.sum(a * w2_ref[...], axis=1, keepdims=True) + b2_ref[...]
    bid = b_ref[0, 0, :].reshape(ROWS, 1)
    gcols = lax.broadcasted_iota(jnp.int32, (ROWS, G), 1)
    onehot = (bid == gcols).astype(jnp.float32)
    contrib = jnp.sum(o * onehot, axis=0, keepdims=True)

    @pl.when(i == 0)
    def _():
        out_ref[...] = contrib

    @pl.when(i > 0)
    def _():
        out_ref[...] = out_ref[...] + contrib

    @pl.when(i == NROW_BLK - 1)
    def _():
        out_ref[...] = _sigmoid(out_ref[...])


def _readout(h, batch3, ow1, ob1, w2r, ob2):
    return pl.pallas_call(
        _out_body,
        grid=(NROW_BLK,),
        in_specs=[
            pl.BlockSpec((ROWS, H), lambda i: (i, 0)),
            pl.BlockSpec((1, 1, ROWS), lambda i: (i, 0, 0)),
            pl.BlockSpec((H, H // 2), lambda i: (0, 0)),
            pl.BlockSpec((1, H // 2), lambda i: (0, 0)),
            pl.BlockSpec((1, H // 2), lambda i: (0, 0)),
            pl.BlockSpec((1, 1), lambda i: (0, 0)),
        ],
        out_specs=pl.BlockSpec((1, G), lambda i: (0, 0)),
        out_shape=jax.ShapeDtypeStruct((1, G), jnp.float32),
    )(h, batch3, ow1, ob1, w2r, ob2)


def kernel(z, pos, batch, emb, mlp_w1, mlp_b1, mlp_w2, mlp_b2,
           cf_lin1, cf_lin2_w, cf_lin2_b, lin_w, lin_b,
           out_w1, out_b1, out_w2, out_b2):
    batch_i = batch.astype(jnp.int32)
    obnd = jnp.searchsorted(batch_i, jnp.arange(G + 1, dtype=jnp.int32),
                            side="left").astype(jnp.int32)
    seg_o = obnd[:-1]
    seg_s = obnd[1:] - seg_o
    cnt = seg_s * jnp.maximum(seg_s - 1, 0)
    offs = jnp.concatenate([jnp.zeros((1,), jnp.int32),
                            jnp.cumsum(cnt).astype(jnp.int32)])
    offs_col = jnp.full((384,), 3.4e7, jnp.float32)
    offs_col = offs_col.at[:G + 1].set(offs.astype(jnp.float32))
    offs_col = offs_col.reshape(384, 1)
    tbl_t = jnp.zeros((8, G), jnp.float32)
    tbl_t = tbl_t.at[0].set(offs[:G].astype(jnp.float32))
    tbl_t = tbl_t.at[1].set(seg_o.astype(jnp.float32))
    tbl_t = tbl_t.at[2].set(jnp.maximum(seg_s - 1, 1).astype(jnp.float32))

    srcf3, dstf3, vm3 = _pair_enum(offs_col, tbl_t)
    srcf = srcf3.reshape(E_MAX)
    dstf = dstf3.reshape(E_MAX)
    d2 = _sc_edge_d2(pos[:, 0], pos[:, 1], pos[:, 2], srcf, dstf)
    d3 = d2.reshape(NE_BLK, 1, ET)
    c3 = vm3
    bounds = jnp.arange(SC_TILES + 1, dtype=jnp.int32) * SRPT
    es = jnp.searchsorted(srcf, bounds[:-1], side="left").astype(jnp.int32)
    ee = jnp.searchsorted(srcf, bounds[1:], side="left").astype(jnp.int32)
    cstart = es // CK
    cend = jnp.where(ee > es, (ee + CK - 1) // CK, cstart)
    cb = jnp.zeros((SC_TILES, 16), jnp.int32)
    cb = cb.at[:, 0].set(cstart).at[:, 1].set(cend)
    z3 = z.astype(jnp.int32).reshape(NROW_BLK, 1, ROWS)
    batch3 = batch.astype(jnp.int32).reshape(NROW_BLK, 1, ROWS)

    emb_p = jnp.zeros((128, H), jnp.float32).at[:100].set(emb)
    w1_p = jnp.pad(mlp_w1, ((0, 0), (0, 64 - NG), (0, 0)))

    h, xp = _embed(z3, emb_p, cf_lin1[0])
    for t in range(T):
        w_e = _edge_filter(d3, c3, w1_p[t], mlp_b1[t].reshape(1, F),
                           mlp_w2[t], mlp_b2[t].reshape(1, F))
        aggp = _sc_edge_messages(xp, w_e, srcf, dstf, cb)
        cf1n = cf_lin1[t + 1] if t + 1 < T else cf_lin1[T - 1]
        h, xp = _update(aggp, h, cf_lin2_w[t], cf_lin2_b[t].reshape(1, H),
                        lin_w[t], lin_b[t].reshape(1, H), cf1n)

    pooled = _readout(h, batch3, out_w1, out_b1.reshape(1, H // 2),
                      out_w2.reshape(1, H // 2), out_b2.reshape(1, 1))
    return pooled.reshape(G)


# speedup vs baseline: 1.0164x; 1.0164x over previous
import functools
import math

import jax
import jax.numpy as jnp
from jax import lax
from jax.experimental import pallas as pl
from jax.experimental.pallas import tpu as pltpu
from jax.experimental.pallas import tpu_sc as plsc

N = 4096
G = 256
H = 128
F = 128
NG = 50
T = 6
CUTOFF = 5.0
E_MAX = 131072

LOG2 = math.log(2.0)
_DELTA = CUTOFF / (NG - 1)
_COEFF = -0.5 / _DELTA**2

ROWS = 512
ET = 512
NROW_BLK = N // ROWS
NE_BLK = E_MAX // ET

SC_TILES = 32
EPW = E_MAX // SC_TILES
CK = 128
NCH = EPW // CK


def _ssp(x):
    return jnp.maximum(x, 0.0) + jnp.log(1.0 + jnp.exp(-jnp.abs(x))) - LOG2


def _sigmoid(x):
    return 1.0 / (1.0 + jnp.exp(-x))


def _embed_body(z_ref, emb_ref, cf1_ref, h_ref, xp_ref):
    z = z_ref[0, 0, :].reshape(ROWS, 1)
    cols = lax.broadcasted_iota(jnp.int32, (ROWS, 128), 1)
    onehot = (z == cols).astype(jnp.float32)
    h = jnp.dot(onehot, emb_ref[...], preferred_element_type=jnp.float32)
    h_ref[...] = h
    xp_ref[...] = jnp.dot(h, cf1_ref[...], preferred_element_type=jnp.float32)


def _embed(z3, emb_p, cf1_0):
    return pl.pallas_call(
        _embed_body,
        grid=(NROW_BLK,),
        in_specs=[
            pl.BlockSpec((1, 1, ROWS), lambda i: (i, 0, 0)),
            pl.BlockSpec((128, H), lambda i: (0, 0)),
            pl.BlockSpec((H, H), lambda i: (0, 0)),
        ],
        out_specs=[
            pl.BlockSpec((ROWS, H), lambda i: (i, 0)),
            pl.BlockSpec((ROWS, H), lambda i: (i, 0)),
        ],
        out_shape=[
            jax.ShapeDtypeStruct((N, H), jnp.float32),
            jax.ShapeDtypeStruct((N, H), jnp.float32),
        ],
    )(z3, emb_p, cf1_0)


def _pair_body(offs_ref, tbl_ref, srcf_ref, dstf_ref, vm_ref):
    i = pl.program_id(0)
    k = i * ET + lax.broadcasted_iota(jnp.int32, (1, ET), 1)
    kf = k.astype(jnp.float32)
    cmp = (offs_ref[...] <= kf).astype(jnp.float32)
    ge = jnp.sum(cmp, axis=0, keepdims=True).astype(jnp.int32) - 1
    vm = ge < G
    gec = jnp.clip(ge, 0, G - 1)
    onehot = (gec == lax.broadcasted_iota(jnp.int32, (G, ET), 0))
    look = jnp.dot(tbl_ref[...], onehot.astype(jnp.float32),
                   preferred_element_type=jnp.float32,
                   precision=lax.Precision.HIGHEST)
    offs_e = look[0:1, :].astype(jnp.int32)
    sego_e = look[1:2, :].astype(jnp.int32)
    sm1f = look[2:3, :]
    sm1 = sm1f.astype(jnp.int32)
    local = k - offs_e
    a = jnp.floor(local.astype(jnp.float32) / sm1f).astype(jnp.int32)
    for _ in range(2):
        r = local - a * sm1
        a = a - (r < 0).astype(jnp.int32) + (r >= sm1).astype(jnp.int32)
    r = local - a * sm1
    b = r + (r >= a).astype(jnp.int32)
    srcf_ref[...] = jnp.where(vm, sego_e + a, N).reshape(1, 1, ET)
    dstf_ref[...] = jnp.where(vm, sego_e + b, 0).reshape(1, 1, ET)
    vm_ref[...] = vm.astype(jnp.float32).reshape(1, 1, ET)


def _pair_enum(offs_col, tbl_t):
    return pl.pallas_call(
        _pair_body,
        grid=(NE_BLK,),
        in_specs=[
            pl.BlockSpec((384, 1), lambda i: (0, 0)),
            pl.BlockSpec((8, G), lambda i: (0, 0)),
        ],
        out_specs=[
            pl.BlockSpec((1, 1, ET), lambda i: (i, 0, 0)),
            pl.BlockSpec((1, 1, ET), lambda i: (i, 0, 0)),
            pl.BlockSpec((1, 1, ET), lambda i: (i, 0, 0)),
        ],
        out_shape=[
            jax.ShapeDtypeStruct((NE_BLK, 1, ET), jnp.int32),
            jax.ShapeDtypeStruct((NE_BLK, 1, ET), jnp.int32),
            jax.ShapeDtypeStruct((NE_BLK, 1, ET), jnp.float32),
        ],
    )(offs_col, tbl_t)


def _scg_body(px_hbm, py_hbm, pz_hbm, srcf_hbm, dstf_hbm, d2_hbm,
              sv, dv, siv, ax, ay, az, bx, by, bz, d2v, sem):
    c = lax.axis_index("c")
    s = lax.axis_index("s")
    tile = c * 16 + s
    base0 = tile * EPW
    nmax = jnp.full((16,), N - 1, jnp.int32)

    def chunk(j, carry):
        base = base0 + j * CK
        pltpu.sync_copy(srcf_hbm.at[pl.ds(base, CK)], sv)
        pltpu.sync_copy(dstf_hbm.at[pl.ds(base, CK)], dv)

        def clip_row(q, cc):
            sl = pl.ds(q * 16, 16)
            siv[sl] = jnp.minimum(sv[sl], nmax)
            return cc

        lax.fori_loop(0, CK // 16, clip_row, 0)
        cps = [pltpu.async_copy(px_hbm.at[siv], ax, sem),
               pltpu.async_copy(py_hbm.at[siv], ay, sem),
               pltpu.async_copy(pz_hbm.at[siv], az, sem),
               pltpu.async_copy(px_hbm.at[dv], bx, sem),
               pltpu.async_copy(py_hbm.at[dv], by, sem),
               pltpu.async_copy(pz_hbm.at[dv], bz, sem)]
        for cp in cps:
            cp.wait()

        def d2_row(q, cc):
            sl = pl.ds(q * 16, 16)
            dx = ax[sl] - bx[sl]
            dy = ay[sl] - by[sl]
            dz = az[sl] - bz[sl]
            d2v[sl] = dx * dx + dy * dy + dz * dz
            return cc

        lax.fori_loop(0, CK // 16, d2_row, 0)
        pltpu.sync_copy(d2v, d2_hbm.at[pl.ds(base, CK)])
        return carry

    lax.fori_loop(0, NCH, chunk, 0)


def _sc_edge_d2(px, py, pz, srcf, dstf):
    mesh = plsc.VectorSubcoreMesh(core_axis_name="c", subcore_axis_name="s")
    fn = pl.kernel(
        _scg_body,
        out_type=jax.ShapeDtypeStruct((E_MAX,), jnp.float32),
        mesh=mesh,
        scratch_types=[
            pltpu.VMEM((CK,), jnp.int32),
            pltpu.VMEM((CK,), jnp.int32),
            pltpu.VMEM((CK,), jnp.int32),
            pltpu.VMEM((CK,), jnp.float32),
            pltpu.VMEM((CK,), jnp.float32),
            pltpu.VMEM((CK,), jnp.float32),
            pltpu.VMEM((CK,), jnp.float32),
            pltpu.VMEM((CK,), jnp.float32),
            pltpu.VMEM((CK,), jnp.float32),
            pltpu.VMEM((CK,), jnp.float32),
            pltpu.SemaphoreType.DMA,
        ],
    )
    return fn(px, py, pz, srcf, dstf)


def _filter_body(d2_ref, vm_ref, w1_ref, b1_ref, w2_ref, b2_ref, w_ref):
    d2 = d2_ref[0, 0, :].reshape(ET, 1)
    d = jnp.sqrt(d2 + 1e-12)
    valid = vm_ref[0, 0, :].reshape(ET, 1) * \
        (d2 < CUTOFF**2).astype(jnp.float32)
    c = 0.5 * (jnp.cos(d * (math.pi / CUTOFF)) + 1.0) * valid
    k = lax.broadcasted_iota(jnp.int32, (ET, 64), 1)
    off = k.astype(jnp.float32) * _DELTA
    rbf = jnp.exp(_COEFF * (d - off) ** 2)
    rbf = jnp.where(k < NG, rbf, 0.0)
    a = _ssp(jnp.dot(rbf, w1_ref[...], preferred_element_type=jnp.float32)
             + b1_ref[...])
    w = jnp.dot(a, w2_ref[...], preferred_element_type=jnp.float32) + b2_ref[...]
    w_ref[...] = w * c


def _edge_filter(d3, c3, w1_p, b1, w2, b2):
    return pl.pallas_call(
        _filter_body,
        grid=(NE_BLK,),
        in_specs=[
            pl.BlockSpec((1, 1, ET), lambda i: (i, 0, 0)),
            pl.BlockSpec((1, 1, ET), lambda i: (i, 0, 0)),
            pl.BlockSpec((64, F), lambda i: (0, 0)),
            pl.BlockSpec((1, F), lambda i: (0, 0)),
            pl.BlockSpec((F, H), lambda i: (0, 0)),
            pl.BlockSpec((1, H), lambda i: (0, 0)),
        ],
        out_specs=pl.BlockSpec((ET, H), lambda i: (i, 0)),
        out_shape=jax.ShapeDtypeStruct((E_MAX, H), jnp.float32),
    )(d3, c3, w1_p, b1, w2, b2)


SRPT = N // SC_TILES


def _sc_body(xp_hbm, w_hbm, srcf_hbm, dstf_hbm, cb_hbm, out_hbm,
             cbv, sv0, sv1, dv0, dv1, wv0, wv1, xv0, xv1, agg,
             semi0, semi1, semw0, semw1, semg0, semg1):
    c = lax.axis_index("c")
    s = lax.axis_index("s")
    tile = c * 16 + s
    lo = tile * SRPT
    zero = jnp.zeros((16,), jnp.float32)
    svs = (sv0, sv1)
    dvs = (dv0, dv1)
    wvs = (wv0, wv1)
    xvs = (xv0, xv1)
    semis = (semi0, semi1)
    semws = (semw0, semw1)
    semgs = (semg0, semg1)

    def zrow(r, carry):
        for g in range(H // 16):
            agg[pl.ds(r * H + g * 16, 16)] = zero
        return carry

    lax.fori_loop(0, SRPT, zrow, 0)
    ntail = jnp.full((16,), N, jnp.int32)
    sv0[pl.ds(CK, 16)] = ntail
    sv1[pl.ds(CK, 16)] = ntail

    pltpu.sync_copy(cb_hbm.at[tile], cbv)
    cbl = cbv[...]
    c0 = cbl[0]
    c1 = cbl[1]
    nch = c1 - c0

    def issue_lin(j, b):
        base = (c0 + j) * CK
        pltpu.async_copy(dstf_hbm.at[pl.ds(base, CK)], dvs[b], semis[b])
        pltpu.async_copy(srcf_hbm.at[pl.ds(base, CK)],
                         svs[b].at[pl.ds(0, CK)], semis[b])
        pltpu.async_copy(w_hbm.at[pl.ds(base, CK)], wvs[b], semws[b])

    def wait_idx(b):
        pltpu.make_async_copy(dstf_hbm.at[pl.ds(0, CK)], dvs[b],
                              semis[b]).wait()
        pltpu.make_async_copy(srcf_hbm.at[pl.ds(0, CK)],
                              svs[b].at[pl.ds(0, CK)], semis[b]).wait()

    def issue_gather(b):
        pltpu.async_copy(xp_hbm.at[dvs[b]], xvs[b], semgs[b])

    def wait_wg(b):
        pltpu.make_async_copy(w_hbm.at[pl.ds(0, CK)], wvs[b],
                              semws[b]).wait()
        pltpu.make_async_copy(xp_hbm.at[dvs[b]], xvs[b], semgs[b]).wait()

    def compute(b):
        sv, wv, xv = svs[b], wvs[b], xvs[b]

        def edge(e, cc):
            row = sv[pl.ds(e, 16)][0] - lo

            @pl.when((row >= 0) & (row < SRPT))
            def _():
                rb = row * H
                for g in range(H // 16):
                    sl = pl.ds(g * 16, 16)
                    sla = pl.ds(rb + g * 16, 16)
                    v = xv[e, sl] * wv[e, sl]
                    agg[sla] = agg[sla] + v

            return cc

        lax.fori_loop(0, CK, edge, 0)

    @pl.when(nch > 0)
    def _():
        issue_lin(0, 0)
        wait_idx(0)
        issue_gather(0)

    def pair(p, carry):
        j0 = 2 * p
        j1 = j0 + 1

        @pl.when(j0 < nch)
        def _():
            @pl.when(j1 < nch)
            def _():
                issue_lin(j1, 1)

            wait_wg(0)

            @pl.when(j1 < nch)
            def _():
                wait_idx(1)
                issue_gather(1)

            compute(0)

        @pl.when(j1 < nch)
        def _():
            @pl.when(j1 + 1 < nch)
            def _():
                issue_lin(j1 + 1, 0)

            wait_wg(1)

            @pl.when(j1 + 1 < nch)
            def _():
                wait_idx(0)
                issue_gather(0)

            compute(1)

        return carry

    lax.fori_loop(0, (nch + 1) // 2, pair, 0)
    pltpu.sync_copy(agg, out_hbm.at[pl.ds(lo * H, SRPT * H)])


def _sc_edge_messages(xp, w_e, srcf, dstf, cb):
    mesh = plsc.VectorSubcoreMesh(core_axis_name="c", subcore_axis_name="s")
    fn = pl.kernel(
        _sc_body,
        out_type=jax.ShapeDtypeStruct((N * H,), jnp.float32),
        mesh=mesh,
        scratch_types=[
            pltpu.VMEM((16,), jnp.int32),
            pltpu.VMEM((CK + 16,), jnp.int32),
            pltpu.VMEM((CK + 16,), jnp.int32),
            pltpu.VMEM((CK,), jnp.int32),
            pltpu.VMEM((CK,), jnp.int32),
            pltpu.VMEM((CK, H), jnp.float32),
            pltpu.VMEM((CK, H), jnp.float32),
            pltpu.VMEM((CK, H), jnp.float32),
            pltpu.VMEM((CK, H), jnp.float32),
            pltpu.VMEM((SRPT * H,), jnp.float32),
            pltpu.SemaphoreType.DMA,
            pltpu.SemaphoreType.DMA,
            pltpu.SemaphoreType.DMA,
            pltpu.SemaphoreType.DMA,
            pltpu.SemaphoreType.DMA,
            pltpu.SemaphoreType.DMA,
        ],
    )
    return fn(xp, w_e, srcf, dstf, cb).reshape(N, H)


def _update_body(agg_ref, h_ref, cf2w_ref, cf2b_ref, linw_ref, linb_ref,
                 cf1n_ref, hn_ref, xpn_ref):
    agg = agg_ref[...]
    xc = _ssp(jnp.dot(agg, cf2w_ref[...], preferred_element_type=jnp.float32)
              + cf2b_ref[...])
    xc = jnp.dot(xc, linw_ref[...], preferred_element_type=jnp.float32) \
        + linb_ref[...]
    hn = h_ref[...] + xc
    hn_ref[...] = hn
    xpn_ref[...] = jnp.dot(hn, cf1n_ref[...], preferred_element_type=jnp.float32)


def _update(aggp, h, cf2w, cf2b, linw, linb, cf1n):
    return pl.pallas_call(
        _update_body,
        grid=(NROW_BLK,),
        in_specs=[
            pl.BlockSpec((ROWS, H), lambda i: (i, 0)),
            pl.BlockSpec((ROWS, H), lambda i: (i, 0)),
            pl.BlockSpec((H, F), lambda i: (0, 0)),
            pl.BlockSpec((1, F), lambda i: (0, 0)),
            pl.BlockSpec((H, H), lambda i: (0, 0)),
            pl.BlockSpec((1, H), lambda i: (0, 0)),
            pl.BlockSpec((H, H), lambda i: (0, 0)),
        ],
        out_specs=[
            pl.BlockSpec((ROWS, H), lambda i: (i, 0)),
            pl.BlockSpec((ROWS, H), lambda i: (i, 0)),
        ],
        out_shape=[
            jax.ShapeDtypeStruct((N, H), jnp.float32),
            jax.ShapeDtypeStruct((N, H), jnp.float32),
        ],
    )(aggp, h, cf2w, cf2b, linw, linb, cf1n)


def _out_body(h_ref, b_ref, w1_ref, b1_ref, w2_ref, b2_ref, out_ref):
    i = pl.program_id(0)
    a = _ssp(jnp.dot(h_ref[...], w1_ref[...], preferred_element_type=jnp.float32)
             + b1_ref[...])
    o = jnp.sum(a * w2_ref[...], axis=1, keepdims=True) + b2_ref[...]
    bid = b_ref[0, 0, :].reshape(ROWS, 1)
    gcols = lax.broadcasted_iota(jnp.int32, (ROWS, G), 1)
    onehot = (bid == gcols).astype(jnp.float32)
    contrib = jnp.sum(o * onehot, axis=0, keepdims=True)

    @pl.when(i == 0)
    def _():
        out_ref[...] = contrib

    @pl.when(i > 0)
    def _():
        out_ref[...] = out_ref[...] + contrib

    @pl.when(i == NROW_BLK - 1)
    def _():
        out_ref[...] = _sigmoid(out_ref[...])


def _readout(h, batch3, ow1, ob1, w2r, ob2):
    return pl.pallas_call(
        _out_body,
        grid=(NROW_BLK,),
        in_specs=[
            pl.BlockSpec((ROWS, H), lambda i: (i, 0)),
            pl.BlockSpec((1, 1, ROWS), lambda i: (i, 0, 0)),
            pl.BlockSpec((H, H // 2), lambda i: (0, 0)),
            pl.BlockSpec((1, H // 2), lambda i: (0, 0)),
            pl.BlockSpec((1, H // 2), lambda i: (0, 0)),
            pl.BlockSpec((1, 1), lambda i: (0, 0)),
        ],
        out_specs=pl.BlockSpec((1, G), lambda i: (0, 0)),
        out_shape=jax.ShapeDtypeStruct((1, G), jnp.float32),
    )(h, batch3, ow1, ob1, w2r, ob2)


def kernel(z, pos, batch, emb, mlp_w1, mlp_b1, mlp_w2, mlp_b2,
           cf_lin1, cf_lin2_w, cf_lin2_b, lin_w, lin_b,
           out_w1, out_b1, out_w2, out_b2):
    batch_i = batch.astype(jnp.int32)
    obnd = jnp.searchsorted(batch_i, jnp.arange(G + 1, dtype=jnp.int32),
                            side="left").astype(jnp.int32)
    seg_o = obnd[:-1]
    seg_s = obnd[1:] - seg_o
    cnt = seg_s * jnp.maximum(seg_s - 1, 0)
    offs = jnp.concatenate([jnp.zeros((1,), jnp.int32),
                            jnp.cumsum(cnt).astype(jnp.int32)])
    offs_col = jnp.full((384,), 3.4e7, jnp.float32)
    offs_col = offs_col.at[:G + 1].set(offs.astype(jnp.float32))
    offs_col = offs_col.reshape(384, 1)
    tbl_t = jnp.zeros((8, G), jnp.float32)
    tbl_t = tbl_t.at[0].set(offs[:G].astype(jnp.float32))
    tbl_t = tbl_t.at[1].set(seg_o.astype(jnp.float32))
    tbl_t = tbl_t.at[2].set(jnp.maximum(seg_s - 1, 1).astype(jnp.float32))

    srcf3, dstf3, vm3 = _pair_enum(offs_col, tbl_t)
    srcf = srcf3.reshape(E_MAX)
    dstf = dstf3.reshape(E_MAX)
    d2 = _sc_edge_d2(pos[:, 0], pos[:, 1], pos[:, 2], srcf, dstf)
    d3 = d2.reshape(NE_BLK, 1, ET)
    c3 = vm3
    bounds = jnp.arange(SC_TILES + 1, dtype=jnp.int32) * SRPT
    es = jnp.searchsorted(srcf, bounds[:-1], side="left").astype(jnp.int32)
    ee = jnp.searchsorted(srcf, bounds[1:], side="left").astype(jnp.int32)
    cstart = es // CK
    cend = jnp.where(ee > es, (ee + CK - 1) // CK, cstart)
    cb = jnp.zeros((SC_TILES, 16), jnp.int32)
    cb = cb.at[:, 0].set(cstart).at[:, 1].set(cend)
    z3 = z.astype(jnp.int32).reshape(NROW_BLK, 1, ROWS)
    batch3 = batch.astype(jnp.int32).reshape(NROW_BLK, 1, ROWS)

    emb_p = jnp.zeros((128, H), jnp.float32).at[:100].set(emb)
    w1_p = jnp.pad(mlp_w1, ((0, 0), (0, 64 - NG), (0, 0)))

    h, xp = _embed(z3, emb_p, cf_lin1[0])
    for t in range(T):
        w_e = _edge_filter(d3, c3, w1_p[t], mlp_b1[t].reshape(1, F),
                           mlp_w2[t], mlp_b2[t].reshape(1, F))
        aggp = _sc_edge_messages(xp, w_e, srcf, dstf, cb)
        cf1n = cf_lin1[t + 1] if t + 1 < T else cf_lin1[T - 1]
        h, xp = _update(aggp, h, cf_lin2_w[t], cf_lin2_b[t].reshape(1, H),
                        lin_w[t], lin_b[t].reshape(1, H), cf1n)

    pooled = _readout(h, batch3, out_w1, out_b1.reshape(1, H // 2),
                      out_w2.reshape(1, H // 2), out_b2.reshape(1, 1))
    return pooled.reshape(G)


# speedup vs baseline: 1.0198x; 1.0033x over previous
import functools
import math

import jax
import jax.numpy as jnp
from jax import lax
from jax.experimental import pallas as pl
from jax.experimental.pallas import tpu as pltpu
from jax.experimental.pallas import tpu_sc as plsc

N = 4096
G = 256
H = 128
F = 128
NG = 50
T = 6
CUTOFF = 5.0
E_MAX = 131072

LOG2 = math.log(2.0)
_DELTA = CUTOFF / (NG - 1)
_COEFF = -0.5 / _DELTA**2

ROWS = 512
ET = 512
NROW_BLK = N // ROWS
NE_BLK = E_MAX // ET

SC_TILES = 32
EPW = E_MAX // SC_TILES
CK = 128
NCH = EPW // CK


def _ssp(x):
    return jnp.maximum(x, 0.0) + jnp.log(1.0 + jnp.exp(-jnp.abs(x))) - LOG2


def _sigmoid(x):
    return 1.0 / (1.0 + jnp.exp(-x))


def _embed_body(z_ref, emb_ref, cf1_ref, h_ref, xp_ref):
    z = z_ref[0, 0, :].reshape(ROWS, 1)
    cols = lax.broadcasted_iota(jnp.int32, (ROWS, 128), 1)
    onehot = (z == cols).astype(jnp.float32)
    h = jnp.dot(onehot, emb_ref[...], preferred_element_type=jnp.float32)
    h_ref[...] = h
    xp_ref[...] = jnp.dot(h, cf1_ref[...], preferred_element_type=jnp.float32)


def _embed(z3, emb_p, cf1_0):
    return pl.pallas_call(
        _embed_body,
        grid=(NROW_BLK,),
        in_specs=[
            pl.BlockSpec((1, 1, ROWS), lambda i: (i, 0, 0)),
            pl.BlockSpec((128, H), lambda i: (0, 0)),
            pl.BlockSpec((H, H), lambda i: (0, 0)),
        ],
        out_specs=[
            pl.BlockSpec((ROWS, H), lambda i: (i, 0)),
            pl.BlockSpec((ROWS, H), lambda i: (i, 0)),
        ],
        out_shape=[
            jax.ShapeDtypeStruct((N, H), jnp.float32),
            jax.ShapeDtypeStruct((N, H), jnp.float32),
        ],
    )(z3, emb_p, cf1_0)


def _pair_body(offs_ref, tbl_ref, srcf_ref, dstf_ref, vm_ref):
    i = pl.program_id(0)
    k = i * ET + lax.broadcasted_iota(jnp.int32, (1, ET), 1)
    kf = k.astype(jnp.float32)
    cmp = (offs_ref[...] <= kf).astype(jnp.float32)
    ge = jnp.sum(cmp, axis=0, keepdims=True).astype(jnp.int32) - 1
    vm = ge < G
    gec = jnp.clip(ge, 0, G - 1)
    onehot = (gec == lax.broadcasted_iota(jnp.int32, (G, ET), 0))
    look = jnp.dot(tbl_ref[...], onehot.astype(jnp.float32),
                   preferred_element_type=jnp.float32,
                   precision=lax.Precision.HIGHEST)
    offs_e = look[0:1, :].astype(jnp.int32)
    sego_e = look[1:2, :].astype(jnp.int32)
    sm1f = look[2:3, :]
    sm1 = sm1f.astype(jnp.int32)
    local = k - offs_e
    a = jnp.floor(local.astype(jnp.float32) / sm1f).astype(jnp.int32)
    for _ in range(2):
        r = local - a * sm1
        a = a - (r < 0).astype(jnp.int32) + (r >= sm1).astype(jnp.int32)
    r = local - a * sm1
    b = r + (r >= a).astype(jnp.int32)
    srcf_ref[...] = jnp.where(vm, sego_e + a, N).reshape(1, 1, ET)
    dstf_ref[...] = jnp.where(vm, sego_e + b, 0).reshape(1, 1, ET)
    vm_ref[...] = vm.astype(jnp.float32).reshape(1, 1, ET)


def _pair_enum(offs_col, tbl_t):
    return pl.pallas_call(
        _pair_body,
        grid=(NE_BLK,),
        in_specs=[
            pl.BlockSpec((384, 1), lambda i: (0, 0)),
            pl.BlockSpec((8, G), lambda i: (0, 0)),
        ],
        out_specs=[
            pl.BlockSpec((1, 1, ET), lambda i: (i, 0, 0)),
            pl.BlockSpec((1, 1, ET), lambda i: (i, 0, 0)),
            pl.BlockSpec((1, 1, ET), lambda i: (i, 0, 0)),
        ],
        out_shape=[
            jax.ShapeDtypeStruct((NE_BLK, 1, ET), jnp.int32),
            jax.ShapeDtypeStruct((NE_BLK, 1, ET), jnp.int32),
            jax.ShapeDtypeStruct((NE_BLK, 1, ET), jnp.float32),
        ],
    )(offs_col, tbl_t)


def _scg_body(px_hbm, py_hbm, pz_hbm, srcf_hbm, dstf_hbm, d2_hbm,
              sv, dv, siv, ax, ay, az, bx, by, bz, d2v, sem):
    c = lax.axis_index("c")
    s = lax.axis_index("s")
    tile = c * 16 + s
    base0 = tile * EPW
    nmax = jnp.full((16,), N - 1, jnp.int32)

    def chunk(j, carry):
        base = base0 + j * CK
        pltpu.sync_copy(srcf_hbm.at[pl.ds(base, CK)], sv)
        pltpu.sync_copy(dstf_hbm.at[pl.ds(base, CK)], dv)

        def clip_row(q, cc):
            sl = pl.ds(q * 16, 16)
            siv[sl] = jnp.minimum(sv[sl], nmax)
            return cc

        lax.fori_loop(0, CK // 16, clip_row, 0)
        cps = [pltpu.async_copy(px_hbm.at[siv], ax, sem),
               pltpu.async_copy(py_hbm.at[siv], ay, sem),
               pltpu.async_copy(pz_hbm.at[siv], az, sem),
               pltpu.async_copy(px_hbm.at[dv], bx, sem),
               pltpu.async_copy(py_hbm.at[dv], by, sem),
               pltpu.async_copy(pz_hbm.at[dv], bz, sem)]
        for cp in cps:
            cp.wait()

        def d2_row(q, cc):
            sl = pl.ds(q * 16, 16)
            dx = ax[sl] - bx[sl]
            dy = ay[sl] - by[sl]
            dz = az[sl] - bz[sl]
            d2v[sl] = dx * dx + dy * dy + dz * dz
            return cc

        lax.fori_loop(0, CK // 16, d2_row, 0)
        pltpu.sync_copy(d2v, d2_hbm.at[pl.ds(base, CK)])
        return carry

    lax.fori_loop(0, NCH, chunk, 0)


def _sc_edge_d2(px, py, pz, srcf, dstf):
    mesh = plsc.VectorSubcoreMesh(core_axis_name="c", subcore_axis_name="s")
    fn = pl.kernel(
        _scg_body,
        out_type=jax.ShapeDtypeStruct((E_MAX,), jnp.float32),
        mesh=mesh,
        scratch_types=[
            pltpu.VMEM((CK,), jnp.int32),
            pltpu.VMEM((CK,), jnp.int32),
            pltpu.VMEM((CK,), jnp.int32),
            pltpu.VMEM((CK,), jnp.float32),
            pltpu.VMEM((CK,), jnp.float32),
            pltpu.VMEM((CK,), jnp.float32),
            pltpu.VMEM((CK,), jnp.float32),
            pltpu.VMEM((CK,), jnp.float32),
            pltpu.VMEM((CK,), jnp.float32),
            pltpu.VMEM((CK,), jnp.float32),
            pltpu.SemaphoreType.DMA,
        ],
    )
    return fn(px, py, pz, srcf, dstf)


def _filter_body(d2_ref, vm_ref, w1_ref, b1_ref, w2_ref, b2_ref, w_ref):
    d2 = d2_ref[0, 0, :].reshape(ET, 1)
    d = jnp.sqrt(d2 + 1e-12)
    valid = vm_ref[0, 0, :].reshape(ET, 1) * \
        (d2 < CUTOFF**2).astype(jnp.float32)
    c = 0.5 * (jnp.cos(d * (math.pi / CUTOFF)) + 1.0) * valid
    k = lax.broadcasted_iota(jnp.int32, (ET, 64), 1)
    off = k.astype(jnp.float32) * _DELTA
    rbf = jnp.exp(_COEFF * (d - off) ** 2)
    rbf = jnp.where(k < NG, rbf, 0.0)
    a = _ssp(jnp.dot(rbf, w1_ref[...], preferred_element_type=jnp.float32)
             + b1_ref[...])
    w = jnp.dot(a, w2_ref[...], preferred_element_type=jnp.float32) + b2_ref[...]
    w_ref[...] = w * c


def _edge_filter(d3, c3, w1_p, b1, w2, b2):
    return pl.pallas_call(
        _filter_body,
        grid=(NE_BLK,),
        in_specs=[
            pl.BlockSpec((1, 1, ET), lambda i: (i, 0, 0)),
            pl.BlockSpec((1, 1, ET), lambda i: (i, 0, 0)),
            pl.BlockSpec((64, F), lambda i: (0, 0)),
            pl.BlockSpec((1, F), lambda i: (0, 0)),
            pl.BlockSpec((F, H), lambda i: (0, 0)),
            pl.BlockSpec((1, H), lambda i: (0, 0)),
        ],
        out_specs=pl.BlockSpec((ET, H), lambda i: (i, 0)),
        out_shape=jax.ShapeDtypeStruct((E_MAX, H), jnp.float32),
    )(d3, c3, w1_p, b1, w2, b2)


SRPT = N // SC_TILES


def _sc_body(xp_hbm, w_hbm, srcf_hbm, dstf_hbm, cb_hbm, out_hbm,
             cbv, sv0, sv1, dv0, dv1, wv0, wv1, xv0, xv1, agg,
             semi0, semi1, semw0, semw1, semg0, semg1):
    c = lax.axis_index("c")
    s = lax.axis_index("s")
    tile = c * 16 + s
    lo = tile * SRPT
    zero = jnp.zeros((16,), jnp.float32)
    svs = (sv0, sv1)
    dvs = (dv0, dv1)
    wvs = (wv0, wv1)
    xvs = (xv0, xv1)
    semis = (semi0, semi1)
    semws = (semw0, semw1)
    semgs = (semg0, semg1)

    def zrow(r, carry):
        for g in range(H // 16):
            agg[pl.ds(r * H + g * 16, 16)] = zero
        return carry

    lax.fori_loop(0, SRPT, zrow, 0)
    ntail = jnp.full((16,), N, jnp.int32)
    sv0[pl.ds(CK, 16)] = ntail
    sv1[pl.ds(CK, 16)] = ntail

    pltpu.sync_copy(cb_hbm.at[tile], cbv)
    cbl = cbv[...]
    c0 = cbl[0]
    c1 = cbl[1]
    nch = c1 - c0

    def issue_lin(j, b):
        base = (c0 + j) * CK
        pltpu.async_copy(dstf_hbm.at[pl.ds(base, CK)], dvs[b], semis[b])
        pltpu.async_copy(srcf_hbm.at[pl.ds(base, CK)],
                         svs[b].at[pl.ds(0, CK)], semis[b])
        pltpu.async_copy(w_hbm.at[pl.ds(base, CK)], wvs[b], semws[b])

    def wait_idx(b):
        pltpu.make_async_copy(dstf_hbm.at[pl.ds(0, CK)], dvs[b],
                              semis[b]).wait()
        pltpu.make_async_copy(srcf_hbm.at[pl.ds(0, CK)],
                              svs[b].at[pl.ds(0, CK)], semis[b]).wait()

    def issue_gather(b):
        pltpu.async_copy(xp_hbm.at[dvs[b]], xvs[b], semgs[b])

    def wait_wg(b):
        pltpu.make_async_copy(w_hbm.at[pl.ds(0, CK)], wvs[b],
                              semws[b]).wait()
        pltpu.make_async_copy(xp_hbm.at[dvs[b]], xvs[b], semgs[b]).wait()

    def compute(b):
        sv, wv, xv = svs[b], wvs[b], xvs[b]

        def edge(e, cc):
            row = sv[pl.ds(e, 16)][0] - lo
            ok = (row >= 0) & (row < SRPT)
            rb = jnp.clip(row, 0, SRPT - 1) * H
            mv = jnp.full((16,), jnp.where(ok, 1.0, 0.0), jnp.float32)
            for g in range(H // 16):
                sl = pl.ds(g * 16, 16)
                sla = pl.ds(rb + g * 16, 16)
                v = xv[e, sl] * wv[e, sl]
                plsc.addupdate(agg.at[sla], v * mv)
            return cc

        lax.fori_loop(0, CK, edge, 0, unroll=4)

    @pl.when(nch > 0)
    def _():
        issue_lin(0, 0)
        wait_idx(0)
        issue_gather(0)

    def pair(p, carry):
        j0 = 2 * p
        j1 = j0 + 1

        @pl.when(j0 < nch)
        def _():
            @pl.when(j1 < nch)
            def _():
                issue_lin(j1, 1)

            wait_wg(0)

            @pl.when(j1 < nch)
            def _():
                wait_idx(1)
                issue_gather(1)

            compute(0)

        @pl.when(j1 < nch)
        def _():
            @pl.when(j1 + 1 < nch)
            def _():
                issue_lin(j1 + 1, 0)

            wait_wg(1)

            @pl.when(j1 + 1 < nch)
            def _():
                wait_idx(0)
                issue_gather(0)

            compute(1)

        return carry

    lax.fori_loop(0, (nch + 1) // 2, pair, 0)
    pltpu.sync_copy(agg, out_hbm.at[pl.ds(lo * H, SRPT * H)])


def _sc_edge_messages(xp, w_e, srcf, dstf, cb):
    mesh = plsc.VectorSubcoreMesh(core_axis_name="c", subcore_axis_name="s")
    fn = pl.kernel(
        _sc_body,
        out_type=jax.ShapeDtypeStruct((N * H,), jnp.float32),
        mesh=mesh,
        scratch_types=[
            pltpu.VMEM((16,), jnp.int32),
            pltpu.VMEM((CK + 16,), jnp.int32),
            pltpu.VMEM((CK + 16,), jnp.int32),
            pltpu.VMEM((CK,), jnp.int32),
            pltpu.VMEM((CK,), jnp.int32),
            pltpu.VMEM((CK, H), jnp.float32),
            pltpu.VMEM((CK, H), jnp.float32),
            pltpu.VMEM((CK, H), jnp.float32),
            pltpu.VMEM((CK, H), jnp.float32),
            pltpu.VMEM((SRPT * H,), jnp.float32),
            pltpu.SemaphoreType.DMA,
            pltpu.SemaphoreType.DMA,
            pltpu.SemaphoreType.DMA,
            pltpu.SemaphoreType.DMA,
            pltpu.SemaphoreType.DMA,
            pltpu.SemaphoreType.DMA,
        ],
    )
    return fn(xp, w_e, srcf, dstf, cb).reshape(N, H)


def _update_body(agg_ref, h_ref, cf2w_ref, cf2b_ref, linw_ref, linb_ref,
                 cf1n_ref, hn_ref, xpn_ref):
    agg = agg_ref[...]
    xc = _ssp(jnp.dot(agg, cf2w_ref[...], preferred_element_type=jnp.float32)
              + cf2b_ref[...])
    xc = jnp.dot(xc, linw_ref[...], preferred_element_type=jnp.float32) \
        + linb_ref[...]
    hn = h_ref[...] + xc
    hn_ref[...] = hn
    xpn_ref[...] = jnp.dot(hn, cf1n_ref[...], preferred_element_type=jnp.float32)


def _update(aggp, h, cf2w, cf2b, linw, linb, cf1n):
    return pl.pallas_call(
        _update_body,
        grid=(NROW_BLK,),
        in_specs=[
            pl.BlockSpec((ROWS, H), lambda i: (i, 0)),
            pl.BlockSpec((ROWS, H), lambda i: (i, 0)),
            pl.BlockSpec((H, F), lambda i: (0, 0)),
            pl.BlockSpec((1, F), lambda i: (0, 0)),
            pl.BlockSpec((H, H), lambda i: (0, 0)),
            pl.BlockSpec((1, H), lambda i: (0, 0)),
            pl.BlockSpec((H, H), lambda i: (0, 0)),
        ],
        out_specs=[
            pl.BlockSpec((ROWS, H), lambda i: (i, 0)),
            pl.BlockSpec((ROWS, H), lambda i: (i, 0)),
        ],
        out_shape=[
            jax.ShapeDtypeStruct((N, H), jnp.float32),
            jax.ShapeDtypeStruct((N, H), jnp.float32),
        ],
    )(aggp, h, cf2w, cf2b, linw, linb, cf1n)


def _out_body(h_ref, b_ref, w1_ref, b1_ref, w2_ref, b2_ref, out_ref):
    i = pl.program_id(0)
    a = _ssp(jnp.dot(h_ref[...], w1_ref[...], preferred_element_type=jnp.float32)
             + b1_ref[...])
    o = jnp.sum(a * w2_ref[...], axis=1, keepdims=True) + b2_ref[...]
    bid = b_ref[0, 0, :].reshape(ROWS, 1)
    gcols = lax.broadcasted_iota(jnp.int32, (ROWS, G), 1)
    onehot = (bid == gcols).astype(jnp.float32)
    contrib = jnp.sum(o * onehot, axis=0, keepdims=True)

    @pl.when(i == 0)
    def _():
        out_ref[...] = contrib

    @pl.when(i > 0)
    def _():
        out_ref[...] = out_ref[...] + contrib

    @pl.when(i == NROW_BLK - 1)
    def _():
        out_ref[...] = _sigmoid(out_ref[...])


def _readout(h, batch3, ow1, ob1, w2r, ob2):
    return pl.pallas_call(
        _out_body,
        grid=(NROW_BLK,),
        in_specs=[
            pl.BlockSpec((ROWS, H), lambda i: (i, 0)),
            pl.BlockSpec((1, 1, ROWS), lambda i: (i, 0, 0)),
            pl.BlockSpec((H, H // 2), lambda i: (0, 0)),
            pl.BlockSpec((1, H // 2), lambda i: (0, 0)),
            pl.BlockSpec((1, H // 2), lambda i: (0, 0)),
            pl.BlockSpec((1, 1), lambda i: (0, 0)),
        ],
        out_specs=pl.BlockSpec((1, G), lambda i: (0, 0)),
        out_shape=jax.ShapeDtypeStruct((1, G), jnp.float32),
    )(h, batch3, ow1, ob1, w2r, ob2)


def kernel(z, pos, batch, emb, mlp_w1, mlp_b1, mlp_w2, mlp_b2,
           cf_lin1, cf_lin2_w, cf_lin2_b, lin_w, lin_b,
           out_w1, out_b1, out_w2, out_b2):
    batch_i = batch.astype(jnp.int32)
    obnd = jnp.searchsorted(batch_i, jnp.arange(G + 1, dtype=jnp.int32),
                            side="left").astype(jnp.int32)
    seg_o = obnd[:-1]
    seg_s = obnd[1:] - seg_o
    cnt = seg_s * jnp.maximum(seg_s - 1, 0)
    offs = jnp.concatenate([jnp.zeros((1,), jnp.int32),
                            jnp.cumsum(cnt).astype(jnp.int32)])
    offs_col = jnp.full((384,), 3.4e7, jnp.float32)
    offs_col = offs_col.at[:G + 1].set(offs.astype(jnp.float32))
    offs_col = offs_col.reshape(384, 1)
    tbl_t = jnp.zeros((8, G), jnp.float32)
    tbl_t = tbl_t.at[0].set(offs[:G].astype(jnp.float32))
    tbl_t = tbl_t.at[1].set(seg_o.astype(jnp.float32))
    tbl_t = tbl_t.at[2].set(jnp.maximum(seg_s - 1, 1).astype(jnp.float32))

    srcf3, dstf3, vm3 = _pair_enum(offs_col, tbl_t)
    srcf = srcf3.reshape(E_MAX)
    dstf = dstf3.reshape(E_MAX)
    d2 = _sc_edge_d2(pos[:, 0], pos[:, 1], pos[:, 2], srcf, dstf)
    d3 = d2.reshape(NE_BLK, 1, ET)
    c3 = vm3
    bounds = jnp.arange(SC_TILES + 1, dtype=jnp.int32) * SRPT
    es = jnp.searchsorted(srcf, bounds[:-1], side="left").astype(jnp.int32)
    ee = jnp.searchsorted(srcf, bounds[1:], side="left").astype(jnp.int32)
    cstart = es // CK
    cend = jnp.where(ee > es, (ee + CK - 1) // CK, cstart)
    cb = jnp.zeros((SC_TILES, 16), jnp.int32)
    cb = cb.at[:, 0].set(cstart).at[:, 1].set(cend)
    z3 = z.astype(jnp.int32).reshape(NROW_BLK, 1, ROWS)
    batch3 = batch.astype(jnp.int32).reshape(NROW_BLK, 1, ROWS)

    emb_p = jnp.zeros((128, H), jnp.float32).at[:100].set(emb)
    w1_p = jnp.pad(mlp_w1, ((0, 0), (0, 64 - NG), (0, 0)))

    h, xp = _embed(z3, emb_p, cf_lin1[0])
    for t in range(T):
        w_e = _edge_filter(d3, c3, w1_p[t], mlp_b1[t].reshape(1, F),
                           mlp_w2[t], mlp_b2[t].reshape(1, F))
        aggp = _sc_edge_messages(xp, w_e, srcf, dstf, cb)
        cf1n = cf_lin1[t + 1] if t + 1 < T else cf_lin1[T - 1]
        h, xp = _update(aggp, h, cf_lin2_w[t], cf_lin2_b[t].reshape(1, H),
                        lin_w[t], lin_b[t].reshape(1, H), cf1n)

    pooled = _readout(h, batch3, out_w1, out_b1.reshape(1, H // 2),
                      out_w2.reshape(1, H // 2), out_b2.reshape(1, 1))
    return pooled.reshape(G)


# speedup vs baseline: 1.1055x; 1.0841x over previous
import functools
import math

import jax
import jax.numpy as jnp
from jax import lax
from jax.experimental import pallas as pl
from jax.experimental.pallas import tpu as pltpu
from jax.experimental.pallas import tpu_sc as plsc

N = 4096
G = 256
H = 128
F = 128
NG = 50
T = 6
CUTOFF = 5.0
E_MAX = 131072

LOG2 = math.log(2.0)
_DELTA = CUTOFF / (NG - 1)
_COEFF = -0.5 / _DELTA**2

ROWS = 512
ET = 512
NROW_BLK = N // ROWS
NE_BLK = E_MAX // ET

SC_TILES = 32
EPW = E_MAX // SC_TILES
CK = 128
NCH = EPW // CK


def _ssp(x):
    return jnp.maximum(x, 0.0) + jnp.log(1.0 + jnp.exp(-jnp.abs(x))) - LOG2


def _sigmoid(x):
    return 1.0 / (1.0 + jnp.exp(-x))


def _embed_body(z_ref, emb_ref, cf1_ref, h_ref, xp_ref):
    z = z_ref[0, 0, :].reshape(ROWS, 1)
    cols = lax.broadcasted_iota(jnp.int32, (ROWS, 128), 1)
    onehot = (z == cols).astype(jnp.float32)
    h = jnp.dot(onehot, emb_ref[...], preferred_element_type=jnp.float32)
    h_ref[...] = h
    xp_ref[...] = jnp.dot(h, cf1_ref[...], preferred_element_type=jnp.float32)


def _embed(z3, emb_p, cf1_0):
    return pl.pallas_call(
        _embed_body,
        grid=(NROW_BLK,),
        in_specs=[
            pl.BlockSpec((1, 1, ROWS), lambda i: (i, 0, 0)),
            pl.BlockSpec((128, H), lambda i: (0, 0)),
            pl.BlockSpec((H, H), lambda i: (0, 0)),
        ],
        out_specs=[
            pl.BlockSpec((ROWS, H), lambda i: (i, 0)),
            pl.BlockSpec((ROWS, H), lambda i: (i, 0)),
        ],
        out_shape=[
            jax.ShapeDtypeStruct((N, H), jnp.float32),
            jax.ShapeDtypeStruct((N, H), jnp.float32),
        ],
    )(z3, emb_p, cf1_0)


def _pair_body(offs_ref, tbl_ref, srcf_ref, dstf_ref, vm_ref):
    i = pl.program_id(0)
    k = i * ET + lax.broadcasted_iota(jnp.int32, (1, ET), 1)
    kf = k.astype(jnp.float32)
    cmp = (offs_ref[...] <= kf).astype(jnp.float32)
    ge = jnp.sum(cmp, axis=0, keepdims=True).astype(jnp.int32) - 1
    vm = ge < G
    gec = jnp.clip(ge, 0, G - 1)
    onehot = (gec == lax.broadcasted_iota(jnp.int32, (G, ET), 0))
    look = jnp.dot(tbl_ref[...], onehot.astype(jnp.float32),
                   preferred_element_type=jnp.float32,
                   precision=lax.Precision.HIGHEST)
    offs_e = look[0:1, :].astype(jnp.int32)
    sego_e = look[1:2, :].astype(jnp.int32)
    sm1f = look[2:3, :]
    sm1 = sm1f.astype(jnp.int32)
    local = k - offs_e
    a = jnp.floor(local.astype(jnp.float32) / sm1f).astype(jnp.int32)
    for _ in range(2):
        r = local - a * sm1
        a = a - (r < 0).astype(jnp.int32) + (r >= sm1).astype(jnp.int32)
    r = local - a * sm1
    b = r + (r >= a).astype(jnp.int32)
    srcf_ref[...] = jnp.where(vm, sego_e + a, N).reshape(1, 1, ET)
    dstf_ref[...] = jnp.where(vm, sego_e + b, 0).reshape(1, 1, ET)
    vm_ref[...] = vm.astype(jnp.float32).reshape(1, 1, ET)


def _pair_enum(offs_col, tbl_t):
    return pl.pallas_call(
        _pair_body,
        grid=(NE_BLK,),
        in_specs=[
            pl.BlockSpec((384, 1), lambda i: (0, 0)),
            pl.BlockSpec((8, G), lambda i: (0, 0)),
        ],
        out_specs=[
            pl.BlockSpec((1, 1, ET), lambda i: (i, 0, 0)),
            pl.BlockSpec((1, 1, ET), lambda i: (i, 0, 0)),
            pl.BlockSpec((1, 1, ET), lambda i: (i, 0, 0)),
        ],
        out_shape=[
            jax.ShapeDtypeStruct((NE_BLK, 1, ET), jnp.int32),
            jax.ShapeDtypeStruct((NE_BLK, 1, ET), jnp.int32),
            jax.ShapeDtypeStruct((NE_BLK, 1, ET), jnp.float32),
        ],
    )(offs_col, tbl_t)


def _scg_body(px_hbm, py_hbm, pz_hbm, srcf_hbm, dstf_hbm, nc_hbm, d2_hbm,
              ncv, sv, dv, siv, ax, ay, az, bx, by, bz, d2v, sem):
    c = lax.axis_index("c")
    s = lax.axis_index("s")
    tile = c * 16 + s
    base0 = tile * EPW
    nmax = jnp.full((16,), N - 1, jnp.int32)
    pltpu.sync_copy(nc_hbm, ncv)
    nct = ncv[...][0]
    nloc = jnp.clip(nct - tile * NCH, 0, NCH)

    def chunk(j, carry):
        base = base0 + j * CK
        pltpu.sync_copy(srcf_hbm.at[pl.ds(base, CK)], sv)
        pltpu.sync_copy(dstf_hbm.at[pl.ds(base, CK)], dv)

        def clip_row(q, cc):
            sl = pl.ds(q * 16, 16)
            siv[sl] = jnp.minimum(sv[sl], nmax)
            return cc

        lax.fori_loop(0, CK // 16, clip_row, 0)
        cps = [pltpu.async_copy(px_hbm.at[siv], ax, sem),
               pltpu.async_copy(py_hbm.at[siv], ay, sem),
               pltpu.async_copy(pz_hbm.at[siv], az, sem),
               pltpu.async_copy(px_hbm.at[dv], bx, sem),
               pltpu.async_copy(py_hbm.at[dv], by, sem),
               pltpu.async_copy(pz_hbm.at[dv], bz, sem)]
        for cp in cps:
            cp.wait()

        def d2_row(q, cc):
            sl = pl.ds(q * 16, 16)
            dx = ax[sl] - bx[sl]
            dy = ay[sl] - by[sl]
            dz = az[sl] - bz[sl]
            d2v[sl] = dx * dx + dy * dy + dz * dz
            return cc

        lax.fori_loop(0, CK // 16, d2_row, 0)
        pltpu.sync_copy(d2v, d2_hbm.at[pl.ds(base, CK)])
        return carry

    lax.fori_loop(0, nloc, chunk, 0)


def _sc_edge_d2(px, py, pz, srcf, dstf, ncb):
    mesh = plsc.VectorSubcoreMesh(core_axis_name="c", subcore_axis_name="s")
    fn = pl.kernel(
        _scg_body,
        out_type=jax.ShapeDtypeStruct((E_MAX,), jnp.float32),
        mesh=mesh,
        scratch_types=[
            pltpu.VMEM((16,), jnp.int32),
            pltpu.VMEM((CK,), jnp.int32),
            pltpu.VMEM((CK,), jnp.int32),
            pltpu.VMEM((CK,), jnp.int32),
            pltpu.VMEM((CK,), jnp.float32),
            pltpu.VMEM((CK,), jnp.float32),
            pltpu.VMEM((CK,), jnp.float32),
            pltpu.VMEM((CK,), jnp.float32),
            pltpu.VMEM((CK,), jnp.float32),
            pltpu.VMEM((CK,), jnp.float32),
            pltpu.VMEM((CK,), jnp.float32),
            pltpu.SemaphoreType.DMA,
        ],
    )
    return fn(px, py, pz, srcf, dstf, ncb)


def _filter_body(d2_ref, vm_ref, w1_ref, b1_ref, w2_ref, b2_ref, w_ref):
    d2 = d2_ref[0, 0, :].reshape(ET, 1)
    d = jnp.sqrt(d2 + 1e-12)
    valid = vm_ref[0, 0, :].reshape(ET, 1) * \
        (d2 < CUTOFF**2).astype(jnp.float32)
    c = 0.5 * (jnp.cos(d * (math.pi / CUTOFF)) + 1.0) * valid
    k = lax.broadcasted_iota(jnp.int32, (ET, 64), 1)
    off = k.astype(jnp.float32) * _DELTA
    rbf = jnp.exp(_COEFF * (d - off) ** 2)
    rbf = jnp.where(k < NG, rbf, 0.0)
    a = _ssp(jnp.dot(rbf, w1_ref[...], preferred_element_type=jnp.float32)
             + b1_ref[...])
    w = jnp.dot(a, w2_ref[...], preferred_element_type=jnp.float32) + b2_ref[...]
    w_ref[...] = w * c


def _edge_filter(d3, c3, w1_p, b1, w2, b2):
    return pl.pallas_call(
        _filter_body,
        grid=(NE_BLK,),
        in_specs=[
            pl.BlockSpec((1, 1, ET), lambda i: (i, 0, 0)),
            pl.BlockSpec((1, 1, ET), lambda i: (i, 0, 0)),
            pl.BlockSpec((64, F), lambda i: (0, 0)),
            pl.BlockSpec((1, F), lambda i: (0, 0)),
            pl.BlockSpec((F, H), lambda i: (0, 0)),
            pl.BlockSpec((1, H), lambda i: (0, 0)),
        ],
        out_specs=pl.BlockSpec((ET, H), lambda i: (i, 0)),
        out_shape=jax.ShapeDtypeStruct((E_MAX, H), jnp.float32),
    )(d3, c3, w1_p, b1, w2, b2)


SRPT = N // SC_TILES


def _sc_body(xp_hbm, w_hbm, srcf_hbm, dstf_hbm, cb_hbm, out_hbm,
             cbv, sv0, sv1, dv0, dv1, wv0, wv1, xv0, xv1, agg,
             semi0, semi1, semw0, semw1, semg0, semg1):
    c = lax.axis_index("c")
    s = lax.axis_index("s")
    tile = c * 16 + s
    lo = tile * SRPT
    zero = jnp.zeros((16,), jnp.float32)
    svs = (sv0, sv1)
    dvs = (dv0, dv1)
    wvs = (wv0, wv1)
    xvs = (xv0, xv1)
    semis = (semi0, semi1)
    semws = (semw0, semw1)
    semgs = (semg0, semg1)

    def zrow(r, carry):
        for g in range(H // 16):
            agg[pl.ds(r * H + g * 16, 16)] = zero
        return carry

    lax.fori_loop(0, SRPT, zrow, 0)
    ntail = jnp.full((16,), N, jnp.int32)
    sv0[pl.ds(CK, 16)] = ntail
    sv1[pl.ds(CK, 16)] = ntail

    pltpu.sync_copy(cb_hbm.at[tile], cbv)
    cbl = cbv[...]
    c0 = cbl[0]
    c1 = cbl[1]
    nch = c1 - c0

    def issue_lin(j, b):
        base = (c0 + j) * CK
        pltpu.async_copy(dstf_hbm.at[pl.ds(base, CK)], dvs[b], semis[b])
        pltpu.async_copy(srcf_hbm.at[pl.ds(base, CK)],
                         svs[b].at[pl.ds(0, CK)], semis[b])
        pltpu.async_copy(w_hbm.at[pl.ds(base, CK)], wvs[b], semws[b])

    def wait_idx(b):
        pltpu.make_async_copy(dstf_hbm.at[pl.ds(0, CK)], dvs[b],
                              semis[b]).wait()
        pltpu.make_async_copy(srcf_hbm.at[pl.ds(0, CK)],
                              svs[b].at[pl.ds(0, CK)], semis[b]).wait()

    def issue_gather(b):
        pltpu.async_copy(xp_hbm.at[dvs[b]], xvs[b], semgs[b])

    def wait_wg(b):
        pltpu.make_async_copy(w_hbm.at[pl.ds(0, CK)], wvs[b],
                              semws[b]).wait()
        pltpu.make_async_copy(xp_hbm.at[dvs[b]], xvs[b], semgs[b]).wait()

    def compute(b):
        sv, wv, xv = svs[b], wvs[b], xvs[b]

        def edge(e, cc):
            row = sv[pl.ds(e, 16)][0] - lo
            ok = (row >= 0) & (row < SRPT)
            rb = jnp.clip(row, 0, SRPT - 1) * H
            mv = jnp.full((16,), jnp.where(ok, 1.0, 0.0), jnp.float32)
            for g in range(H // 16):
                sl = pl.ds(g * 16, 16)
                sla = pl.ds(rb + g * 16, 16)
                v = xv[e, sl] * wv[e, sl]
                plsc.addupdate(agg.at[sla], v * mv)
            return cc

        lax.fori_loop(0, CK, edge, 0, unroll=4)

    @pl.when(nch > 0)
    def _():
        issue_lin(0, 0)
        wait_idx(0)
        issue_gather(0)

    def pair(p, carry):
        j0 = 2 * p
        j1 = j0 + 1

        @pl.when(j0 < nch)
        def _():
            @pl.when(j1 < nch)
            def _():
                issue_lin(j1, 1)

            wait_wg(0)

            @pl.when(j1 < nch)
            def _():
                wait_idx(1)
                issue_gather(1)

            compute(0)

        @pl.when(j1 < nch)
        def _():
            @pl.when(j1 + 1 < nch)
            def _():
                issue_lin(j1 + 1, 0)

            wait_wg(1)

            @pl.when(j1 + 1 < nch)
            def _():
                wait_idx(0)
                issue_gather(0)

            compute(1)

        return carry

    lax.fori_loop(0, (nch + 1) // 2, pair, 0)
    pltpu.sync_copy(agg, out_hbm.at[pl.ds(lo * H, SRPT * H)])


def _sc_edge_messages(xp, w_e, srcf, dstf, cb):
    mesh = plsc.VectorSubcoreMesh(core_axis_name="c", subcore_axis_name="s")
    fn = pl.kernel(
        _sc_body,
        out_type=jax.ShapeDtypeStruct((N * H,), jnp.float32),
        mesh=mesh,
        scratch_types=[
            pltpu.VMEM((16,), jnp.int32),
            pltpu.VMEM((CK + 16,), jnp.int32),
            pltpu.VMEM((CK + 16,), jnp.int32),
            pltpu.VMEM((CK,), jnp.int32),
            pltpu.VMEM((CK,), jnp.int32),
            pltpu.VMEM((CK, H), jnp.float32),
            pltpu.VMEM((CK, H), jnp.float32),
            pltpu.VMEM((CK, H), jnp.float32),
            pltpu.VMEM((CK, H), jnp.float32),
            pltpu.VMEM((SRPT * H,), jnp.float32),
            pltpu.SemaphoreType.DMA,
            pltpu.SemaphoreType.DMA,
            pltpu.SemaphoreType.DMA,
            pltpu.SemaphoreType.DMA,
            pltpu.SemaphoreType.DMA,
            pltpu.SemaphoreType.DMA,
        ],
    )
    return fn(xp, w_e, srcf, dstf, cb).reshape(N, H)


def _update_body(agg_ref, h_ref, cf2w_ref, cf2b_ref, linw_ref, linb_ref,
                 cf1n_ref, hn_ref, xpn_ref):
    agg = agg_ref[...]
    xc = _ssp(jnp.dot(agg, cf2w_ref[...], preferred_element_type=jnp.float32)
              + cf2b_ref[...])
    xc = jnp.dot(xc, linw_ref[...], preferred_element_type=jnp.float32) \
        + linb_ref[...]
    hn = h_ref[...] + xc
    hn_ref[...] = hn
    xpn_ref[...] = jnp.dot(hn, cf1n_ref[...], preferred_element_type=jnp.float32)


def _update(aggp, h, cf2w, cf2b, linw, linb, cf1n):
    return pl.pallas_call(
        _update_body,
        grid=(NROW_BLK,),
        in_specs=[
            pl.BlockSpec((ROWS, H), lambda i: (i, 0)),
            pl.BlockSpec((ROWS, H), lambda i: (i, 0)),
            pl.BlockSpec((H, F), lambda i: (0, 0)),
            pl.BlockSpec((1, F), lambda i: (0, 0)),
            pl.BlockSpec((H, H), lambda i: (0, 0)),
            pl.BlockSpec((1, H), lambda i: (0, 0)),
            pl.BlockSpec((H, H), lambda i: (0, 0)),
        ],
        out_specs=[
            pl.BlockSpec((ROWS, H), lambda i: (i, 0)),
            pl.BlockSpec((ROWS, H), lambda i: (i, 0)),
        ],
        out_shape=[
            jax.ShapeDtypeStruct((N, H), jnp.float32),
            jax.ShapeDtypeStruct((N, H), jnp.float32),
        ],
    )(aggp, h, cf2w, cf2b, linw, linb, cf1n)


def _out_body(h_ref, b_ref, w1_ref, b1_ref, w2_ref, b2_ref, out_ref):
    i = pl.program_id(0)
    a = _ssp(jnp.dot(h_ref[...], w1_ref[...], preferred_element_type=jnp.float32)
             + b1_ref[...])
    o = jnp.sum(a * w2_ref[...], axis=1, keepdims=True) + b2_ref[...]
    bid = b_ref[0, 0, :].reshape(ROWS, 1)
    gcols = lax.broadcasted_iota(jnp.int32, (ROWS, G), 1)
    onehot = (bid == gcols).astype(jnp.float32)
    contrib = jnp.sum(o * onehot, axis=0, keepdims=True)

    @pl.when(i == 0)
    def _():
        out_ref[...] = contrib

    @pl.when(i > 0)
    def _():
        out_ref[...] = out_ref[...] + contrib

    @pl.when(i == NROW_BLK - 1)
    def _():
        out_ref[...] = _sigmoid(out_ref[...])


def _readout(h, batch3, ow1, ob1, w2r, ob2):
    return pl.pallas_call(
        _out_body,
        grid=(NROW_BLK,),
        in_specs=[
            pl.BlockSpec((ROWS, H), lambda i: (i, 0)),
            pl.BlockSpec((1, 1, ROWS), lambda i: (i, 0, 0)),
            pl.BlockSpec((H, H // 2), lambda i: (0, 0)),
            pl.BlockSpec((1, H // 2), lambda i: (0, 0)),
            pl.BlockSpec((1, H // 2), lambda i: (0, 0)),
            pl.BlockSpec((1, 1), lambda i: (0, 0)),
        ],
        out_specs=pl.BlockSpec((1, G), lambda i: (0, 0)),
        out_shape=jax.ShapeDtypeStruct((1, G), jnp.float32),
    )(h, batch3, ow1, ob1, w2r, ob2)


def kernel(z, pos, batch, emb, mlp_w1, mlp_b1, mlp_w2, mlp_b2,
           cf_lin1, cf_lin2_w, cf_lin2_b, lin_w, lin_b,
           out_w1, out_b1, out_w2, out_b2):
    batch_i = batch.astype(jnp.int32)
    obnd = jnp.searchsorted(batch_i, jnp.arange(G + 1, dtype=jnp.int32),
                            side="left").astype(jnp.int32)
    seg_o = obnd[:-1]
    seg_s = obnd[1:] - seg_o
    cnt = seg_s * jnp.maximum(seg_s - 1, 0)
    offs = jnp.concatenate([jnp.zeros((1,), jnp.int32),
                            jnp.cumsum(cnt).astype(jnp.int32)])
    offs_col = jnp.full((384,), 3.4e7, jnp.float32)
    offs_col = offs_col.at[:G + 1].set(offs.astype(jnp.float32))
    offs_col = offs_col.reshape(384, 1)
    tbl_t = jnp.zeros((8, G), jnp.float32)
    tbl_t = tbl_t.at[0].set(offs[:G].astype(jnp.float32))
    tbl_t = tbl_t.at[1].set(seg_o.astype(jnp.float32))
    tbl_t = tbl_t.at[2].set(jnp.maximum(seg_s - 1, 1).astype(jnp.float32))

    srcf3, dstf3, vm3 = _pair_enum(offs_col, tbl_t)
    srcf = srcf3.reshape(E_MAX)
    dstf = dstf3.reshape(E_MAX)
    nch_tot = (offs[G] + CK - 1) // CK
    ncb = jnp.broadcast_to(nch_tot.reshape(1), (16,)).astype(jnp.int32)
    d2 = _sc_edge_d2(pos[:, 0], pos[:, 1], pos[:, 2], srcf, dstf, ncb)
    d3 = d2.reshape(NE_BLK, 1, ET)
    c3 = vm3
    bounds = jnp.arange(SC_TILES + 1, dtype=jnp.int32) * SRPT
    es = jnp.searchsorted(srcf, bounds[:-1], side="left").astype(jnp.int32)
    ee = jnp.searchsorted(srcf, bounds[1:], side="left").astype(jnp.int32)
    cstart = es // CK
    cend = jnp.where(ee > es, (ee + CK - 1) // CK, cstart)
    cb = jnp.zeros((SC_TILES, 16), jnp.int32)
    cb = cb.at[:, 0].set(cstart).at[:, 1].set(cend)
    z3 = z.astype(jnp.int32).reshape(NROW_BLK, 1, ROWS)
    batch3 = batch.astype(jnp.int32).reshape(NROW_BLK, 1, ROWS)

    emb_p = jnp.zeros((128, H), jnp.float32).at[:100].set(emb)
    w1_p = jnp.pad(mlp_w1, ((0, 0), (0, 64 - NG), (0, 0)))

    h, xp = _embed(z3, emb_p, cf_lin1[0])
    for t in range(T):
        w_e = _edge_filter(d3, c3, w1_p[t], mlp_b1[t].reshape(1, F),
                           mlp_w2[t], mlp_b2[t].reshape(1, F))
        aggp = _sc_edge_messages(xp, w_e, srcf, dstf, cb)
        cf1n = cf_lin1[t + 1] if t + 1 < T else cf_lin1[T - 1]
        h, xp = _update(aggp, h, cf_lin2_w[t], cf_lin2_b[t].reshape(1, H),
                        lin_w[t], lin_b[t].reshape(1, H), cf1n)

    pooled = _readout(h, batch3, out_w1, out_b1.reshape(1, H // 2),
                      out_w2.reshape(1, H // 2), out_b2.reshape(1, 1))
    return pooled.reshape(G)
